# baseline (speedup 1.0000x reference)
import functools

import jax
import jax.numpy as jnp
from jax import lax
from jax.experimental import pallas as pl
from jax.experimental.pallas import tpu as pltpu
from jax.experimental.pallas import tpu_sc as plsc

_CK = 512


def _dist_kernel(x_ref, wp_ref, bp_ref, emb_ref,
                 idx_ref, gidx_ref, loss_ref,
                 p_s, a_s, best_s, bidx_s, *, moff):
    k = pl.program_id(0)
    nk = pl.num_programs(0)

    @pl.when(k == 0)
    def _():
        x = x_ref[...]
        p = jnp.dot(x, wp_ref[0], preferred_element_type=jnp.float32)
        p = p + bp_ref[0]
        p_s[...] = p
        ones = jnp.ones((1, p.shape[1]), jnp.float32)
        a_s[...] = lax.dot_general(ones, p * p, (((1,), (1,)), ((), ())),
                                   preferred_element_type=jnp.float32)
        best_s[...] = jnp.full_like(best_s, jnp.inf)
        bidx_s[...] = jnp.zeros_like(bidx_s)

    e = emb_ref[0]
    b_col = jnp.sum(e * e, axis=1, keepdims=True)
    c_t = lax.dot_general(e, p_s[...], (((1,), (1,)), ((), ())),
                          preferred_element_type=jnp.float32)
    d_t = (a_s[...] + b_col) - 2.0 * c_t
    cur_min = jnp.min(d_t, axis=0, keepdims=True)
    hit = d_t == cur_min
    iota = lax.broadcasted_iota(jnp.int32, d_t.shape, 0)
    cur_arg = jnp.min(jnp.where(hit, iota, jnp.int32(2**30)),
                      axis=0, keepdims=True) + k * _CK
    better = cur_min < best_s[...]
    best_s[...] = jnp.where(better, cur_min, best_s[...])
    bidx_s[...] = jnp.where(better, cur_arg, bidx_s[...])

    @pl.when(k == nk - 1)
    def _():
        loss_ref[0, 0] = jnp.sum(best_s[...])
        idx_ref[...] = bidx_s[...]
        gidx_ref[...] = bidx_s[...] + moff


def _outproj_kernel(*refs):
    qs = refs[:8]
    wo_ref, bo_ref, o_ref = refs[8], refs[9], refs[10]
    m = pl.program_id(0)
    for c in range(8):
        @pl.when(m == c)
        def _():
            o = jnp.dot(qs[c][...], wo_ref[0],
                        preferred_element_type=jnp.float32)
            o_ref[...] = o + bo_ref[0]


def _sc_gather(tbl_hbm, gidx_hbm, out_hbm, idx_v, rows_v, sem, *, nc, rpw):
    wid = lax.axis_index("s") * nc + lax.axis_index("c")
    base = wid * rpw
    pltpu.sync_copy(gidx_hbm.at[pl.ds(base, rpw)], idx_v)
    pltpu.async_copy(tbl_hbm.at[idx_v], rows_v, sem).wait()
    pltpu.sync_copy(rows_v, out_hbm.at[pl.ds(base, rpw)])


def kernel(slots, Wp, bp, emb, Wo, bo):
    B, N, D = slots.shape
    M = Wp.shape[0]
    blk = D // M
    R = B * N
    V = emb.shape[1]
    x2d = slots.reshape(R, D)
    nk = V // _CK
    bp3 = bp.reshape(M, 1, blk)
    bo3 = bo.reshape(M, 1, blk)
    tbl = emb.reshape(M * V, blk)

    info = plsc.get_sparse_core_info()
    nc, ns = info.num_cores, info.num_subcores
    rpw = R // (nc * ns)

    idx_rows, qs, losses = [], [], []
    for m in range(M):
        idx_m, gidx_m, loss_m = pl.pallas_call(
            functools.partial(_dist_kernel, moff=m * V),
            grid=(nk,),
            in_specs=[
                pl.BlockSpec((R, blk), lambda k, m=m: (0, m)),
                pl.BlockSpec((1, blk, blk), lambda k, m=m: (m, 0, 0)),
                pl.BlockSpec((1, 1, blk), lambda k, m=m: (m, 0, 0)),
                pl.BlockSpec((1, _CK, blk), lambda k, m=m: (m, k, 0)),
            ],
            out_specs=[
                pl.BlockSpec((1, R), lambda k: (0, 0)),
                pl.BlockSpec((1, R), lambda k: (0, 0)),
                pl.BlockSpec(memory_space=pltpu.SMEM),
            ],
            out_shape=[
                jax.ShapeDtypeStruct((1, R), jnp.int32),
                jax.ShapeDtypeStruct((1, R), jnp.int32),
                jax.ShapeDtypeStruct((1, 1), jnp.float32),
            ],
            scratch_shapes=[
                pltpu.VMEM((R, blk), jnp.float32),
                pltpu.VMEM((1, R), jnp.float32),
                pltpu.VMEM((1, R), jnp.float32),
                pltpu.VMEM((1, R), jnp.int32),
            ],
            compiler_params=pltpu.CompilerParams(
                dimension_semantics=("arbitrary",)),
        )(x2d, Wp, bp3, emb)
        idx_rows.append(idx_m)
        losses.append(loss_m)

        gather = functools.partial(
            pl.kernel,
            out_type=jax.ShapeDtypeStruct((R, blk), jnp.float32),
            mesh=plsc.VectorSubcoreMesh(core_axis_name="c",
                                        subcore_axis_name="s"),
            scratch_types=[
                pltpu.VMEM((rpw,), jnp.int32),
                pltpu.VMEM((rpw, blk), jnp.float32),
                pltpu.SemaphoreType.DMA,
            ],
        )(functools.partial(_sc_gather, nc=nc, rpw=rpw))
        qs.append(gather(tbl, gidx_m.reshape(R)))

    out2d = pl.pallas_call(
        _outproj_kernel,
        grid=(M,),
        in_specs=[pl.BlockSpec((R, blk), lambda m: (0, 0))] * M + [
            pl.BlockSpec((1, blk, blk), lambda m: (m, 0, 0)),
            pl.BlockSpec((1, 1, blk), lambda m: (m, 0, 0)),
        ],
        out_specs=pl.BlockSpec((R, blk), lambda m: (0, m)),
        out_shape=jax.ShapeDtypeStruct((R, D), jnp.float32),
        compiler_params=pltpu.CompilerParams(
            dimension_semantics=("arbitrary",)),
    )(*qs, Wo, bo3)

    quantized = out2d.reshape(B, N, D)
    enc = jnp.concatenate(idx_rows, axis=0).T.reshape(B, N, M)
    scale = jnp.float32(1.25 / (8 * 4096 * 256))
    loss = (sum(l.reshape(()) for l in losses)) * scale
    return quantized, loss, enc


# speedup vs baseline: 1.1343x; 1.1343x over previous
import functools

import jax
import jax.numpy as jnp
from jax import lax
from jax.experimental import pallas as pl
from jax.experimental.pallas import tpu as pltpu
from jax.experimental.pallas import tpu_sc as plsc

_CK = 1024


def _dist_kernel(x_ref, wp_ref, bp_ref, emb_ref,
                 idx_ref, gidx_ref, loss_ref,
                 p_s, a_s, best_s, bidx_s, *, moff):
    k = pl.program_id(0)
    nk = pl.num_programs(0)

    @pl.when(k == 0)
    def _():
        x = x_ref[...]
        p = jnp.dot(x, wp_ref[0], preferred_element_type=jnp.float32)
        p = p + bp_ref[0]
        p_s[...] = p
        ones = jnp.ones((1, p.shape[1]), jnp.float32)
        a_s[...] = lax.dot_general(ones, p * p, (((1,), (1,)), ((), ())),
                                   preferred_element_type=jnp.float32)
        best_s[...] = jnp.full_like(best_s, jnp.inf)
        bidx_s[...] = jnp.zeros_like(bidx_s)

    e = emb_ref[0]
    b_col = jnp.sum(e * e, axis=1, keepdims=True)
    c_t = lax.dot_general(e, p_s[...], (((1,), (1,)), ((), ())),
                          preferred_element_type=jnp.float32)
    d_t = (a_s[...] + b_col) - 2.0 * c_t
    cur_min = jnp.min(d_t, axis=0, keepdims=True)
    hit = d_t == cur_min
    iota = lax.broadcasted_iota(jnp.int32, d_t.shape, 0)
    cur_arg = jnp.min(jnp.where(hit, iota, jnp.int32(2**30)),
                      axis=0, keepdims=True) + k * _CK
    better = cur_min < best_s[...]
    best_s[...] = jnp.where(better, cur_min, best_s[...])
    bidx_s[...] = jnp.where(better, cur_arg, bidx_s[...])

    @pl.when(k == nk - 1)
    def _():
        loss_ref[0, 0] = jnp.sum(best_s[...])
        idx_ref[...] = bidx_s[...]
        gidx_ref[...] = bidx_s[...] + moff


def _outproj_kernel(*refs):
    qs = refs[:8]
    wo_ref, bo_ref, o_ref = refs[8], refs[9], refs[10]
    m = pl.program_id(0)
    for c in range(8):
        @pl.when(m == c)
        def _():
            o = jnp.dot(qs[c][...], wo_ref[0],
                        preferred_element_type=jnp.float32)
            o_ref[...] = o + bo_ref[0]


def _sc_gather(tbl_hbm, gidx_hbm, out_hbm, idx_v, rows_v, sem, *, nc, rpw):
    wid = lax.axis_index("s") * nc + lax.axis_index("c")
    base = wid * rpw
    pltpu.sync_copy(gidx_hbm.at[pl.ds(base, rpw)], idx_v)
    pltpu.async_copy(tbl_hbm.at[idx_v], rows_v, sem).wait()
    pltpu.sync_copy(rows_v, out_hbm.at[pl.ds(base, rpw)])


def kernel(slots, Wp, bp, emb, Wo, bo):
    B, N, D = slots.shape
    M = Wp.shape[0]
    blk = D // M
    R = B * N
    V = emb.shape[1]
    x2d = slots.reshape(R, D)
    nk = V // _CK
    bp3 = bp.reshape(M, 1, blk)
    bo3 = bo.reshape(M, 1, blk)
    tbl = emb.reshape(M * V, blk)

    info = plsc.get_sparse_core_info()
    nc, ns = info.num_cores, info.num_subcores
    rpw = R // (nc * ns)

    idx_rows, qs, losses = [], [], []
    for m in range(M):
        idx_m, gidx_m, loss_m = pl.pallas_call(
            functools.partial(_dist_kernel, moff=m * V),
            grid=(nk,),
            in_specs=[
                pl.BlockSpec((R, blk), lambda k, m=m: (0, m)),
                pl.BlockSpec((1, blk, blk), lambda k, m=m: (m, 0, 0)),
                pl.BlockSpec((1, 1, blk), lambda k, m=m: (m, 0, 0)),
                pl.BlockSpec((1, _CK, blk), lambda k, m=m: (m, k, 0)),
            ],
            out_specs=[
                pl.BlockSpec((1, R), lambda k: (0, 0)),
                pl.BlockSpec((1, R), lambda k: (0, 0)),
                pl.BlockSpec(memory_space=pltpu.SMEM),
            ],
            out_shape=[
                jax.ShapeDtypeStruct((1, R), jnp.int32),
                jax.ShapeDtypeStruct((1, R), jnp.int32),
                jax.ShapeDtypeStruct((1, 1), jnp.float32),
            ],
            scratch_shapes=[
                pltpu.VMEM((R, blk), jnp.float32),
                pltpu.VMEM((1, R), jnp.float32),
                pltpu.VMEM((1, R), jnp.float32),
                pltpu.VMEM((1, R), jnp.int32),
            ],
            compiler_params=pltpu.CompilerParams(
                dimension_semantics=("arbitrary",)),
        )(x2d, Wp, bp3, emb)
        idx_rows.append(idx_m)
        losses.append(loss_m)

        gather = functools.partial(
            pl.kernel,
            out_type=jax.ShapeDtypeStruct((R, blk), jnp.float32),
            mesh=plsc.VectorSubcoreMesh(core_axis_name="c",
                                        subcore_axis_name="s"),
            scratch_types=[
                pltpu.VMEM((rpw,), jnp.int32),
                pltpu.VMEM((rpw, blk), jnp.float32),
                pltpu.SemaphoreType.DMA,
            ],
        )(functools.partial(_sc_gather, nc=nc, rpw=rpw))
        qs.append(gather(tbl, gidx_m.reshape(R)))

    out2d = pl.pallas_call(
        _outproj_kernel,
        grid=(M,),
        in_specs=[pl.BlockSpec((R, blk), lambda m: (0, 0))] * M + [
            pl.BlockSpec((1, blk, blk), lambda m: (m, 0, 0)),
            pl.BlockSpec((1, 1, blk), lambda m: (m, 0, 0)),
        ],
        out_specs=pl.BlockSpec((R, blk), lambda m: (0, m)),
        out_shape=jax.ShapeDtypeStruct((R, D), jnp.float32),
        compiler_params=pltpu.CompilerParams(
            dimension_semantics=("arbitrary",)),
    )(*qs, Wo, bo3)

    quantized = out2d.reshape(B, N, D)
    enc = jnp.concatenate(idx_rows, axis=0).T.reshape(B, N, M)
    scale = jnp.float32(1.25 / (8 * 4096 * 256))
    loss = (sum(l.reshape(()) for l in losses)) * scale
    return quantized, loss, enc


# speedup vs baseline: 1.3601x; 1.1991x over previous
import functools

import jax
import jax.numpy as jnp
from jax import lax
from jax.experimental import pallas as pl
from jax.experimental.pallas import tpu as pltpu
from jax.experimental.pallas import tpu_sc as plsc

_CK = 1024


def _dist_kernel(x_ref, wp_ref, bp_ref, emb_ref,
                 idx_ref, gidx_ref, loss_ref,
                 p_s, a_s, best_s, bidx_s, *, moff):
    k = pl.program_id(0)
    nk = pl.num_programs(0)

    @pl.when(k == 0)
    def _():
        x = x_ref[...]
        p = jnp.dot(x, wp_ref[0], preferred_element_type=jnp.float32)
        p = p + bp_ref[0]
        p_s[...] = p
        ones = jnp.ones((1, p.shape[1]), jnp.float32)
        a_s[...] = lax.dot_general(ones, p * p, (((1,), (1,)), ((), ())),
                                   preferred_element_type=jnp.float32)
        best_s[...] = jnp.full_like(best_s, jnp.inf)
        bidx_s[...] = jnp.zeros_like(bidx_s)

    e = emb_ref[0]
    b_col = jnp.sum(e * e, axis=1, keepdims=True)
    c2_t = lax.dot_general(e * jnp.float32(-2.0), p_s[...],
                           (((1,), (1,)), ((), ())),
                           preferred_element_type=jnp.float32)
    d_t = (a_s[...] + b_col) + c2_t
    cur_min = jnp.min(d_t, axis=0, keepdims=True)
    cur_arg = jnp.argmin(d_t, axis=0).astype(jnp.int32)[None, :] + k * _CK
    better = cur_min < best_s[...]
    best_s[...] = jnp.where(better, cur_min, best_s[...])
    bidx_s[...] = jnp.where(better, cur_arg, bidx_s[...])

    @pl.when(k == nk - 1)
    def _():
        loss_ref[0, 0] = jnp.sum(best_s[...])
        idx_ref[...] = bidx_s[...]
        gidx_ref[...] = bidx_s[...] + moff


def _outproj_kernel(*refs):
    qs = refs[:8]
    wo_ref, bo_ref, o_ref = refs[8], refs[9], refs[10]
    m = pl.program_id(0)
    for c in range(8):
        @pl.when(m == c)
        def _():
            o = jnp.dot(qs[c][...], wo_ref[0],
                        preferred_element_type=jnp.float32)
            o_ref[...] = o + bo_ref[0]


def _sc_gather(tbl_hbm, gidx_hbm, out_hbm, idx_v, rows_v, sem, *, nc, rpw):
    wid = lax.axis_index("s") * nc + lax.axis_index("c")
    base = wid * rpw
    pltpu.sync_copy(gidx_hbm.at[pl.ds(base, rpw)], idx_v)
    pltpu.async_copy(tbl_hbm.at[idx_v], rows_v, sem).wait()
    pltpu.sync_copy(rows_v, out_hbm.at[pl.ds(base, rpw)])


def kernel(slots, Wp, bp, emb, Wo, bo):
    B, N, D = slots.shape
    M = Wp.shape[0]
    blk = D // M
    R = B * N
    V = emb.shape[1]
    x2d = slots.reshape(R, D)
    nk = V // _CK
    bp3 = bp.reshape(M, 1, blk)
    bo3 = bo.reshape(M, 1, blk)
    tbl = emb.reshape(M * V, blk)

    info = plsc.get_sparse_core_info()
    nc, ns = info.num_cores, info.num_subcores
    rpw = R // (nc * ns)

    idx_rows, qs, losses = [], [], []
    for m in range(M):
        idx_m, gidx_m, loss_m = pl.pallas_call(
            functools.partial(_dist_kernel, moff=m * V),
            grid=(nk,),
            in_specs=[
                pl.BlockSpec((R, blk), lambda k, m=m: (0, m)),
                pl.BlockSpec((1, blk, blk), lambda k, m=m: (m, 0, 0)),
                pl.BlockSpec((1, 1, blk), lambda k, m=m: (m, 0, 0)),
                pl.BlockSpec((1, _CK, blk), lambda k, m=m: (m, k, 0)),
            ],
            out_specs=[
                pl.BlockSpec((1, R), lambda k: (0, 0)),
                pl.BlockSpec((1, R), lambda k: (0, 0)),
                pl.BlockSpec(memory_space=pltpu.SMEM),
            ],
            out_shape=[
                jax.ShapeDtypeStruct((1, R), jnp.int32),
                jax.ShapeDtypeStruct((1, R), jnp.int32),
                jax.ShapeDtypeStruct((1, 1), jnp.float32),
            ],
            scratch_shapes=[
                pltpu.VMEM((R, blk), jnp.float32),
                pltpu.VMEM((1, R), jnp.float32),
                pltpu.VMEM((1, R), jnp.float32),
                pltpu.VMEM((1, R), jnp.int32),
            ],
            compiler_params=pltpu.CompilerParams(
                dimension_semantics=("arbitrary",)),
        )(x2d, Wp, bp3, emb)
        idx_rows.append(idx_m)
        losses.append(loss_m)

        gather = functools.partial(
            pl.kernel,
            out_type=jax.ShapeDtypeStruct((R, blk), jnp.float32),
            mesh=plsc.VectorSubcoreMesh(core_axis_name="c",
                                        subcore_axis_name="s"),
            scratch_types=[
                pltpu.VMEM((rpw,), jnp.int32),
                pltpu.VMEM((rpw, blk), jnp.float32),
                pltpu.SemaphoreType.DMA,
            ],
        )(functools.partial(_sc_gather, nc=nc, rpw=rpw))
        qs.append(gather(tbl, gidx_m.reshape(R)))

    out2d = pl.pallas_call(
        _outproj_kernel,
        grid=(M,),
        in_specs=[pl.BlockSpec((R, blk), lambda m: (0, 0))] * M + [
            pl.BlockSpec((1, blk, blk), lambda m: (m, 0, 0)),
            pl.BlockSpec((1, 1, blk), lambda m: (m, 0, 0)),
        ],
        out_specs=pl.BlockSpec((R, blk), lambda m: (0, m)),
        out_shape=jax.ShapeDtypeStruct((R, D), jnp.float32),
        compiler_params=pltpu.CompilerParams(
            dimension_semantics=("arbitrary",)),
    )(*qs, Wo, bo3)

    quantized = out2d.reshape(B, N, D)
    enc = jnp.concatenate(idx_rows, axis=0).T.reshape(B, N, M)
    scale = jnp.float32(1.25 / (8 * 4096 * 256))
    loss = (sum(l.reshape(()) for l in losses)) * scale
    return quantized, loss, enc
